# baseline (speedup 1.0000x reference)
import functools

import jax
import jax.numpy as jnp
from jax import lax
from jax.experimental import pallas as pl
from jax.experimental.pallas import tpu as pltpu
from jax.experimental.pallas import tpu_sc as plsc

B, L, D = 4096, 200, 32
N = B * L
NF = 4
SMALL_V = 1000

NC, NS = 2, 16
NW = NC * NS
PER_W = N // NW

IDROWS = 8
SUPER = IDROWS * 128
N_SUPER = PER_W // SUPER

_mesh = plsc.VectorSubcoreMesh(core_axis_name="c", subcore_axis_name="s")
_params = pltpu.CompilerParams(use_tc_tiling_on_sc=False)


def _field_loop(wid, fields, out, idx_v, rows_v, gsem, ssems,
                throttle_ns=0):
    nf = len(fields)

    def one_super(sc, u, first_body):
        row_base = pl.multiple_of(wid * PER_W + sc * SUPER, SUPER)
        idrow_base = pl.multiple_of(row_base // 128, IDROWS)
        for f, (ids_hbm, _, _) in enumerate(fields):
            pltpu.sync_copy(ids_hbm.at[pl.ds(idrow_base, IDROWS)],
                            idx_v.at[f])
        for f, (_, table, col) in enumerate(fields):
            par = (u * nf + f) % 2
            dst = out.at[pl.ds(row_base, SUPER), pl.ds(col, D)]
            drain = pltpu.make_async_copy(rows_v.at[par], dst, ssems[par])
            if first_body is None or u * nf + f >= 2:
                drain.wait()
            else:
                @pl.when(jnp.logical_not(first_body))
                def _():
                    drain.wait()
            gathers = [
                pltpu.async_copy(
                    table.at[idx_v.at[f, j]],
                    rows_v.at[par, pl.ds(j * 128, 128)],
                    gsem,
                )
                for j in range(IDROWS)
            ]
            for cp in gathers:
                cp.wait()
            pltpu.async_copy(rows_v.at[par], dst, ssems[par])
            if throttle_ns:
                pl.delay(throttle_ns)

    def pair_body(p):
        for u in range(2):
            one_super(2 * p + u, u, p == 0)

    pl.loop(0, N_SUPER // 2)(pair_body)
    if N_SUPER % 2:
        one_super(N_SUPER - 1, 0, None)

    tail = out.at[pl.ds(0, SUPER), pl.ds(0, D)]
    pltpu.make_async_copy(rows_v.at[0], tail, ssems[0]).wait()
    pltpu.make_async_copy(rows_v.at[1], tail, ssems[1]).wait()


@functools.partial(
    pl.kernel,
    mesh=_mesh,
    out_type=jax.ShapeDtypeStruct((N, NF * D), jnp.float32),
    scratch_types=[
        pltpu.VMEM((3, IDROWS, 128), jnp.int32),
        pltpu.VMEM((2, SUPER, D), jnp.float32),
        pltpu.VMEM_SHARED((2, SMALL_V, D), jnp.float32),
        pltpu.SemaphoreType.DMA,
        pltpu.SemaphoreType.DMA,
        pltpu.SemaphoreType.DMA,
    ],
    compiler_params=_params,
)
def _sc_embed_scp(shop_ids, cate_ids, price_ids,
                  shop_table, cate_table, price_table,
                  out, idx_v, rows_v, small_sh, gsem, ssem0, ssem1):
    sid = lax.axis_index("s")
    wid = sid * NC + lax.axis_index("c")

    @pl.when(sid == 0)
    def _():
        pltpu.sync_copy(cate_table, small_sh.at[0])
        pltpu.sync_copy(price_table, small_sh.at[1])
    plsc.subcore_barrier()

    fields = (
        (shop_ids, shop_table, 1 * D),
        (cate_ids, small_sh.at[0], 2 * D),
        (price_ids, small_sh.at[1], 3 * D),
    )
    _field_loop(wid, fields, out, idx_v, rows_v, gsem, (ssem0, ssem1),
                throttle_ns=1300)


@functools.partial(
    pl.kernel,
    mesh=_mesh,
    scratch_types=[
        pltpu.VMEM((1, IDROWS, 128), jnp.int32),
        pltpu.VMEM((2, SUPER, D), jnp.float32),
        pltpu.SemaphoreType.DMA,
        pltpu.SemaphoreType.DMA,
        pltpu.SemaphoreType.DMA,
    ],
    compiler_params=_params,
)
def _sc_embed_goods(goods_ids, goods_table, out,
                    idx_v, rows_v, gsem, ssem0, ssem1):
    wid = lax.axis_index("s") * NC + lax.axis_index("c")
    fields = ((goods_ids, goods_table, 0),)
    _field_loop(wid, fields, out, idx_v, rows_v, gsem, (ssem0, ssem1))


def kernel(goods_ids, shop_ids, cate_ids, goods_prices,
           goods_table, shop_table, cate_table, price_table):
    g2, s2, c2, p2 = (x.reshape(N // 128, 128) for x in
                      (goods_ids, shop_ids, cate_ids, goods_prices))
    out_a = _sc_embed_scp(s2, c2, p2, shop_table, cate_table, price_table)
    ref = jax.new_ref(out_a)
    _sc_embed_goods(g2, goods_table, ref)
    return ref[...].reshape(B, L, NF * D)


# speedup vs baseline: 1.0163x; 1.0163x over previous
import functools

import jax
import jax.numpy as jnp
from jax import lax
from jax.experimental import pallas as pl
from jax.experimental.pallas import tpu as pltpu
from jax.experimental.pallas import tpu_sc as plsc

B, L, D = 4096, 200, 32
N = B * L
NF = 4
SMALL_V = 1000

NC, NS = 2, 16
NW = NC * NS
PER_W = N // NW

IDROWS = 8
SUPER = IDROWS * 128
N_SUPER = PER_W // SUPER

_mesh = plsc.VectorSubcoreMesh(core_axis_name="c", subcore_axis_name="s")
_params = pltpu.CompilerParams(use_tc_tiling_on_sc=False)


def _field_loop(wid, fields, out, idx_v, rows_v, gsem, ssems):
    nf = len(fields)

    def one_super(sc, u, first_body):
        row_base = pl.multiple_of(wid * PER_W + sc * SUPER, SUPER)
        idrow_base = pl.multiple_of(row_base // 128, IDROWS)
        for f, (ids_hbm, _, _) in enumerate(fields):
            pltpu.sync_copy(ids_hbm.at[pl.ds(idrow_base, IDROWS)],
                            idx_v.at[f])
        for f, (_, table, col) in enumerate(fields):
            par = (u * nf + f) % 2
            dst = out.at[pl.ds(row_base, SUPER), pl.ds(col, D)]
            drain = pltpu.make_async_copy(rows_v.at[par], dst, ssems[par])
            if first_body is None or u * nf + f >= 2:
                drain.wait()
            else:
                @pl.when(jnp.logical_not(first_body))
                def _():
                    drain.wait()
            gathers = [
                pltpu.async_copy(
                    table.at[idx_v.at[f, j]],
                    rows_v.at[par, pl.ds(j * 128, 128)],
                    gsem,
                )
                for j in range(IDROWS)
            ]
            for cp in gathers:
                cp.wait()
            pltpu.async_copy(rows_v.at[par], dst, ssems[par])

    def pair_body(p):
        for u in range(2):
            one_super(2 * p + u, u, p == 0)

    pl.loop(0, N_SUPER // 2)(pair_body)
    if N_SUPER % 2:
        one_super(N_SUPER - 1, 0, None)

    tail = out.at[pl.ds(0, SUPER), pl.ds(0, D)]
    pltpu.make_async_copy(rows_v.at[0], tail, ssems[0]).wait()
    pltpu.make_async_copy(rows_v.at[1], tail, ssems[1]).wait()


@functools.partial(
    pl.kernel,
    mesh=_mesh,
    out_type=jax.ShapeDtypeStruct((N, NF * D), jnp.float32),
    scratch_types=[
        pltpu.VMEM((3, IDROWS, 128), jnp.int32),
        pltpu.VMEM((2, SUPER, D), jnp.float32),
        pltpu.VMEM_SHARED((2, SMALL_V, D), jnp.float32),
        pltpu.SemaphoreType.DMA,
        pltpu.SemaphoreType.DMA,
        pltpu.SemaphoreType.DMA,
    ],
    compiler_params=_params,
)
def _sc_embed_scp(shop_ids, cate_ids, price_ids,
                  shop_table, cate_table, price_table,
                  out, idx_v, rows_v, small_sh, gsem, ssem0, ssem1):
    sid = lax.axis_index("s")
    wid = sid * NC + lax.axis_index("c")

    @pl.when(sid == 0)
    def _():
        pltpu.sync_copy(cate_table, small_sh.at[0])
        pltpu.sync_copy(price_table, small_sh.at[1])
    plsc.subcore_barrier()

    fields = (
        (shop_ids, shop_table, 1 * D),
        (cate_ids, small_sh.at[0], 2 * D),
        (price_ids, small_sh.at[1], 3 * D),
    )
    _field_loop(wid, fields, out, idx_v, rows_v, gsem, (ssem0, ssem1))


@functools.partial(
    pl.kernel,
    mesh=_mesh,
    scratch_types=[
        pltpu.VMEM((2, IDROWS, 128), jnp.int32),
        pltpu.VMEM((2, SUPER, D), jnp.float32),
        pltpu.SemaphoreType.DMA,
        pltpu.SemaphoreType.DMA,
        pltpu.SemaphoreType.DMA,
    ],
    compiler_params=_params,
)
def _sc_embed_goods(goods_ids, goods_table, out,
                    idx_v, rows_v, gsem, ssem0, ssem1):
    wid = lax.axis_index("s") * NC + lax.axis_index("c")
    ssems = (ssem0, ssem1)

    def stage_ids(s, slot):
        idrow_base = pl.multiple_of((wid * PER_W + s * SUPER) // 128, IDROWS)
        pltpu.sync_copy(goods_ids.at[pl.ds(idrow_base, IDROWS)],
                        idx_v.at[slot])

    def issue(s, slot):
        return [
            pltpu.async_copy(
                goods_table.at[idx_v.at[slot, j]],
                rows_v.at[slot, pl.ds(j * 128, 128)],
                gsem,
            )
            for j in range(IDROWS)
        ]

    def dst_of(s):
        row_base = pl.multiple_of(wid * PER_W + s * SUPER, SUPER)
        return out.at[pl.ds(row_base, SUPER), pl.ds(0, D)]

    def consume(s, slot):
        dst = dst_of(s)
        for _ in range(IDROWS):
            pltpu.make_async_copy(
                goods_table.at[idx_v.at[slot, 0]],
                rows_v.at[slot, pl.ds(0, 128)], gsem).wait()
        pltpu.async_copy(rows_v.at[slot], dst, ssems[slot])

    stage_ids(0, 0)
    issue(0, 0)

    def pair_body(p):
        s0 = 2 * p
        stage_ids(s0 + 1, 1)

        @pl.when(p != 0)
        def _():
            pltpu.make_async_copy(rows_v.at[1], dst_of(s0), ssems[1]).wait()
        issue(s0 + 1, 1)
        consume(s0, 0)
        stage_ids(s0 + 2, 0)
        pltpu.make_async_copy(rows_v.at[0], dst_of(s0), ssems[0]).wait()
        issue(s0 + 2, 0)
        consume(s0 + 1, 1)

    pl.loop(0, N_SUPER // 2)(pair_body)
    consume(N_SUPER - 1, 0)
    pltpu.make_async_copy(rows_v.at[0], dst_of(0), ssems[0]).wait()
    pltpu.make_async_copy(rows_v.at[1], dst_of(0), ssems[1]).wait()


def kernel(goods_ids, shop_ids, cate_ids, goods_prices,
           goods_table, shop_table, cate_table, price_table):
    g2, s2, c2, p2 = (x.reshape(N // 128, 128) for x in
                      (goods_ids, shop_ids, cate_ids, goods_prices))
    out_a = _sc_embed_scp(s2, c2, p2, shop_table, cate_table, price_table)
    ref = jax.new_ref(out_a)
    _sc_embed_goods(g2, goods_table, ref)
    return ref[...].reshape(B, L, NF * D)


# speedup vs baseline: 1.0207x; 1.0043x over previous
import functools

import jax
import jax.numpy as jnp
from jax import lax
from jax.experimental import pallas as pl
from jax.experimental.pallas import tpu as pltpu
from jax.experimental.pallas import tpu_sc as plsc

B, L, D = 4096, 200, 32
N = B * L
NF = 4
SMALL_V = 1000

NC, NS = 2, 16
NW = NC * NS
PER_W = N // NW

IDROWS = 8
SUPER = IDROWS * 128
N_SUPER = PER_W // SUPER

_mesh = plsc.VectorSubcoreMesh(core_axis_name="c", subcore_axis_name="s")
_params = pltpu.CompilerParams(use_tc_tiling_on_sc=False)


def _field_loop(wid, fields, out, idx_v, rows_v, gsem, ssems):
    nf = len(fields)

    def one_super(sc, u, first_body):
        row_base = pl.multiple_of(wid * PER_W + sc * SUPER, SUPER)
        idrow_base = pl.multiple_of(row_base // 128, IDROWS)
        for f, (ids_hbm, _, _) in enumerate(fields):
            pltpu.sync_copy(ids_hbm.at[pl.ds(idrow_base, IDROWS)],
                            idx_v.at[f])
        for f, (_, table, col) in enumerate(fields):
            par = (u * nf + f) % 2
            dst = out.at[pl.ds(row_base, SUPER), pl.ds(col, D)]
            drain = pltpu.make_async_copy(rows_v.at[par], dst, ssems[par])
            if first_body is None or u * nf + f >= 2:
                drain.wait()
            else:
                @pl.when(jnp.logical_not(first_body))
                def _():
                    drain.wait()
            gathers = [
                pltpu.async_copy(
                    table.at[idx_v.at[f, j]],
                    rows_v.at[par, pl.ds(j * 128, 128)],
                    gsem,
                )
                for j in range(IDROWS)
            ]
            for cp in gathers:
                cp.wait()
            pltpu.async_copy(rows_v.at[par], dst, ssems[par])

    def pair_body(p):
        for u in range(2):
            one_super(2 * p + u, u, p == 0)

    pl.loop(0, N_SUPER // 2)(pair_body)
    if N_SUPER % 2:
        one_super(N_SUPER - 1, 0, None)

    tail = out.at[pl.ds(0, SUPER), pl.ds(0, D)]
    pltpu.make_async_copy(rows_v.at[0], tail, ssems[0]).wait()
    pltpu.make_async_copy(rows_v.at[1], tail, ssems[1]).wait()


@functools.partial(
    pl.kernel,
    mesh=_mesh,
    out_type=jax.ShapeDtypeStruct((N, NF * D), jnp.float32),
    scratch_types=[
        pltpu.VMEM((3, IDROWS, 128), jnp.int32),
        pltpu.VMEM((2, SUPER, D), jnp.float32),
        pltpu.VMEM_SHARED((2, SMALL_V, D), jnp.float32),
        pltpu.SemaphoreType.DMA,
        pltpu.SemaphoreType.DMA,
        pltpu.SemaphoreType.DMA,
    ],
    compiler_params=_params,
)
def _sc_embed_scp(shop_ids, cate_ids, price_ids,
                  shop_table, cate_table, price_table,
                  out, idx_v, rows_v, small_sh, gsem, ssem0, ssem1):
    sid = lax.axis_index("s")
    wid = sid * NC + lax.axis_index("c")

    @pl.when(sid == 0)
    def _():
        pltpu.sync_copy(cate_table, small_sh.at[0])
        pltpu.sync_copy(price_table, small_sh.at[1])
    plsc.subcore_barrier()

    fields = (
        (shop_ids, shop_table, 1 * D),
        (cate_ids, small_sh.at[0], 2 * D),
        (price_ids, small_sh.at[1], 3 * D),
    )
    _field_loop(wid, fields, out, idx_v, rows_v, gsem, (ssem0, ssem1))


@functools.partial(
    pl.kernel,
    mesh=_mesh,
    scratch_types=[
        pltpu.VMEM((2, IDROWS, 128), jnp.int32),
        pltpu.VMEM((2, SUPER, D), jnp.float32),
        pltpu.SemaphoreType.DMA,
        pltpu.SemaphoreType.DMA,
        pltpu.SemaphoreType.DMA,
        pltpu.SemaphoreType.DMA,
    ],
    compiler_params=_params,
)
def _sc_embed_goods(goods_ids, goods_table, out,
                    idx_v, rows_v, gsem0, gsem1, ssem0, ssem1):
    wid = lax.axis_index("s") * NC + lax.axis_index("c")
    ssems = (ssem0, ssem1)
    gsems = (gsem0, gsem1)

    def stage_ids(s, slot):
        idrow_base = pl.multiple_of((wid * PER_W + s * SUPER) // 128, IDROWS)
        pltpu.sync_copy(goods_ids.at[pl.ds(idrow_base, IDROWS)],
                        idx_v.at[slot])

    def issue(s, slot):
        return [
            pltpu.async_copy(
                goods_table.at[idx_v.at[slot, j]],
                rows_v.at[slot, pl.ds(j * 128, 128)],
                gsems[slot],
            )
            for j in range(IDROWS)
        ]

    def dst_of(s):
        row_base = pl.multiple_of(wid * PER_W + s * SUPER, SUPER)
        return out.at[pl.ds(row_base, SUPER), pl.ds(0, D)]

    def consume(s, slot):
        dst = dst_of(s)
        for _ in range(IDROWS):
            pltpu.make_async_copy(
                goods_table.at[idx_v.at[slot, 0]],
                rows_v.at[slot, pl.ds(0, 128)], gsems[slot]).wait()
        pltpu.async_copy(rows_v.at[slot], dst, ssems[slot])

    stage_ids(0, 0)
    issue(0, 0)

    def pair_body(p):
        s0 = 2 * p
        stage_ids(s0 + 1, 1)

        @pl.when(p != 0)
        def _():
            pltpu.make_async_copy(rows_v.at[1], dst_of(s0), ssems[1]).wait()
        issue(s0 + 1, 1)
        consume(s0, 0)
        stage_ids(s0 + 2, 0)
        pltpu.make_async_copy(rows_v.at[0], dst_of(s0), ssems[0]).wait()
        issue(s0 + 2, 0)
        consume(s0 + 1, 1)

    pl.loop(0, N_SUPER // 2)(pair_body)
    consume(N_SUPER - 1, 0)
    pltpu.make_async_copy(rows_v.at[0], dst_of(0), ssems[0]).wait()
    pltpu.make_async_copy(rows_v.at[1], dst_of(0), ssems[1]).wait()


def kernel(goods_ids, shop_ids, cate_ids, goods_prices,
           goods_table, shop_table, cate_table, price_table):
    g2, s2, c2, p2 = (x.reshape(N // 128, 128) for x in
                      (goods_ids, shop_ids, cate_ids, goods_prices))
    out_a = _sc_embed_scp(s2, c2, p2, shop_table, cate_table, price_table)
    ref = jax.new_ref(out_a)
    _sc_embed_goods(g2, goods_table, ref)
    return ref[...].reshape(B, L, NF * D)
